# Initial kernel scaffold; baseline (speedup 1.0000x reference)
import jax
import jax.numpy as jnp
from jax.experimental import pallas as pl


def kernel(user_emb, item_emb, user_idx, item_idx):
    raise NotImplementedError("write your pallas kernel here")


# speedup vs baseline: 11.5120x; 11.5120x over previous
import functools
import jax
import jax.numpy as jnp
from jax import lax
from jax.experimental import pallas as pl
from jax.experimental.pallas import tpu as pltpu
from jax.experimental.pallas import tpu_sc as plsc

DQ = 16
NQ = 4
EG = 128
GPC = 8
CH = EG * GPC
NTILES = 16


def _seg_sum_body(nchunks, rpn, src_hbm, dst_hbm, h0, h1, h2, h3, z_hbm,
                  t0, t1, t2, t3, sidx, didx, rows, acc, sem):
    c = lax.axis_index("c")
    s = lax.axis_index("s")

    def do_quarter(h_hbm, t_hbm):
        pltpu.sync_copy(z_hbm, acc.at[pl.ds(s * rpn, rpn)])
        plsc.subcore_barrier()

        def chunk(i, _):
            row0 = (s * nchunks + i) * GPC
            pltpu.sync_copy(src_hbm.at[pl.ds(row0, GPC)], sidx)
            pltpu.sync_copy(dst_hbm.at[pl.ds(row0, GPC)], didx)
            cps = []
            for j in range(GPC):
                cps.append(pltpu.async_copy(
                    h_hbm.at[sidx.at[j]], rows.at[pl.ds(j * EG, EG)], sem))
            for cp in cps:
                cp.wait()
            for j in range(GPC):
                pltpu.sync_copy(rows.at[pl.ds(j * EG, EG)],
                                acc.at[didx.at[j]], add=True)
            return 0

        lax.fori_loop(0, nchunks, chunk, 0)
        plsc.subcore_barrier()
        pltpu.sync_copy(acc.at[pl.ds(s * rpn, rpn)],
                        t_hbm.at[pl.ds(s * rpn, rpn)])

    @pl.when(c == 0)
    def _():
        do_quarter(h0, t0)
        do_quarter(h1, t1)

    @pl.when(c == 1)
    def _():
        do_quarter(h2, t2)
        do_quarter(h3, t3)


def _make_seg_sum(npad, ne_pad):
    nchunks = ne_pad // (NTILES * CH)
    rpn = npad // NTILES
    mesh = plsc.VectorSubcoreMesh(
        core_axis_name="c", subcore_axis_name="s",
        num_cores=2, num_subcores=NTILES)
    return functools.partial(
        pl.kernel,
        mesh=mesh,
        compiler_params=pltpu.CompilerParams(use_tc_tiling_on_sc=False),
        out_type=tuple(jax.ShapeDtypeStruct((npad, DQ), jnp.float32)
                       for _ in range(NQ)),
        scratch_types=[
            pltpu.VMEM((GPC, EG), jnp.int32),
            pltpu.VMEM((GPC, EG), jnp.int32),
            pltpu.VMEM((CH, DQ), jnp.float32),
            pltpu.VMEM_SHARED((npad, DQ), jnp.float32),
            pltpu.SemaphoreType.DMA,
        ],
    )(functools.partial(_seg_sum_body, nchunks, rpn))


def _prep_tc(emb_blk, tdeg_blk, dinv_blk, h0_blk, h1_blk, h2_blk, h3_blk):
    deg = tdeg_blk[:, 0:1]
    dinv = jnp.where(deg > 0.0,
                     jax.lax.rsqrt(jnp.maximum(deg, 1e-12)), 0.0)
    dinvq = jnp.broadcast_to(dinv, (deg.shape[0], DQ))
    dinv_blk[...] = dinvq
    for q, h_blk in enumerate((h0_blk, h1_blk, h2_blk, h3_blk)):
        h_blk[...] = dinvq * emb_blk[:, q * DQ:(q + 1) * DQ]


def _finalize_tc(t0_blk, t1_blk, t2_blk, t3_blk, dinv_blk, tot_blk,
                 h0_blk, h1_blk, h2_blk, h3_blk, out_blk):
    dinvq = dinv_blk[...]
    hs = (h0_blk, h1_blk, h2_blk, h3_blk)
    for q, t_blk in enumerate((t0_blk, t1_blk, t2_blk, t3_blk)):
        g = dinvq * t_blk[...]
        hs[q][...] = dinvq * g
        out_blk[:, q * DQ:(q + 1) * DQ] = tot_blk[:, q * DQ:(q + 1) * DQ] + g


def _last_tc(t0_blk, t1_blk, t2_blk, t3_blk, dinv_blk, tot_blk, out_blk):
    dinvq = dinv_blk[...]
    for q, t_blk in enumerate((t0_blk, t1_blk, t2_blk, t3_blk)):
        out_blk[:, q * DQ:(q + 1) * DQ] = (
            tot_blk[:, q * DQ:(q + 1) * DQ] + dinvq * t_blk[...]) * 0.25


def kernel(user_emb, item_emb, user_idx, item_idx):
    n_users = user_emb.shape[0]
    n_items = item_emb.shape[0]
    n_nodes = n_users + n_items
    n_edges = user_idx.shape[0]
    dim = user_emb.shape[1]

    npad = ((n_nodes + 1 + 127) // 128) * 128
    ne = 2 * n_edges
    ne_pad = ((ne + NTILES * CH - 1) // (NTILES * CH)) * (NTILES * CH)
    rpn = npad // NTILES

    src = jnp.concatenate([user_idx, item_idx + n_users])
    dst = jnp.concatenate([item_idx + n_users, user_idx])
    pad = jnp.full((ne_pad - ne,), n_nodes, dtype=jnp.int32)
    src2 = jnp.concatenate([src, pad]).reshape(ne_pad // EG, EG)
    dst2 = jnp.concatenate([dst, pad]).reshape(ne_pad // EG, EG)

    emb = jnp.concatenate([user_emb, item_emb], axis=0)
    emb = jnp.concatenate(
        [emb, jnp.zeros((npad - n_nodes, dim), jnp.float32)], axis=0)
    onesq = jnp.ones((npad, DQ), jnp.float32)
    z = jnp.zeros((rpn, DQ), jnp.float32)

    seg_sum = _make_seg_sum(npad, ne_pad)

    tdeg = seg_sum(src2, dst2, onesq, onesq, onesq, onesq, z)[0]

    nblk = 16
    rb = npad // nblk
    row_specq = pl.BlockSpec((rb, DQ), lambda i: (i, 0))
    row_spec64 = pl.BlockSpec((rb, dim), lambda i: (i, 0))
    sdq = jax.ShapeDtypeStruct((npad, DQ), jnp.float32)
    dinvq, h0, h1, h2, h3 = pl.pallas_call(
        _prep_tc,
        grid=(nblk,),
        in_specs=[row_spec64, row_specq],
        out_specs=[row_specq] * 5,
        out_shape=[sdq] * 5,
    )(emb, tdeg)

    total = emb
    for layer in range(3):
        t0, t1, t2, t3 = seg_sum(src2, dst2, h0, h1, h2, h3, z)
        if layer < 2:
            h0, h1, h2, h3, total = pl.pallas_call(
                _finalize_tc,
                grid=(nblk,),
                in_specs=[row_specq] * 5 + [row_spec64],
                out_specs=[row_specq] * 4 + [row_spec64],
                out_shape=[sdq] * 4 +
                          [jax.ShapeDtypeStruct((npad, dim), jnp.float32)],
            )(t0, t1, t2, t3, dinvq, total)
        else:
            out = pl.pallas_call(
                _last_tc,
                grid=(nblk,),
                in_specs=[row_specq] * 5 + [row_spec64],
                out_specs=row_spec64,
                out_shape=jax.ShapeDtypeStruct((npad, dim), jnp.float32),
            )(t0, t1, t2, t3, dinvq, total)

    return out[:n_users], out[n_users:n_nodes]


# speedup vs baseline: 18.4487x; 1.6026x over previous
import functools
import jax
import jax.numpy as jnp
from jax import lax
from jax.experimental import pallas as pl
from jax.experimental.pallas import tpu as pltpu
from jax.experimental.pallas import tpu_sc as plsc

DQ = 16
NQ = 4
EG = 128
GPC = 8
CH = EG * GPC
NTILES = 16


def _seg_sum_body(nchunks, rpn, src_hbm, dst_hbm, h0, h1, h2, h3, z_hbm,
                  t0, t1, t2, t3, sidx, didx, rows, acc,
                  gsem0, gsem1, ssem0, ssem1):
    c = lax.axis_index("c")
    s = lax.axis_index("s")
    gsem = (gsem0, gsem1)
    ssem = (ssem0, ssem1)

    def do_quarter(h_hbm, t_hbm):
        pltpu.sync_copy(z_hbm, acc.at[pl.ds(s * rpn, rpn)])
        plsc.subcore_barrier()

        def issue(i, b):
            row0 = (s * nchunks + i) * GPC
            pltpu.sync_copy(src_hbm.at[pl.ds(row0, GPC)], sidx.at[b])
            pltpu.sync_copy(dst_hbm.at[pl.ds(row0, GPC)], didx.at[b])
            for j in range(GPC):
                pltpu.async_copy(h_hbm.at[sidx.at[b].at[j]],
                                 rows.at[b].at[pl.ds(j * EG, EG)], gsem[b])

        def wait_gathers(b):
            for j in range(GPC):
                pltpu.make_async_copy(
                    h_hbm.at[sidx.at[b].at[j]],
                    rows.at[b].at[pl.ds(j * EG, EG)], gsem[b]).wait()

        def fire_scatters(b):
            for j in range(GPC):
                pltpu.async_copy(rows.at[b].at[pl.ds(j * EG, EG)],
                                 acc.at[didx.at[b].at[j]], ssem[b], add=True)

        def drain_scatters(b):
            for j in range(GPC):
                pltpu.make_async_copy(
                    rows.at[b].at[pl.ds(j * EG, EG)],
                    acc.at[didx.at[b].at[j]], ssem[b]).wait()

        issue(0, 0)
        issue(1, 1)

        def body(g, _):
            for b in range(2):
                i = 2 * g + b
                wait_gathers(b)
                fire_scatters(b)
                drain_scatters(b)
                issue(i + 2, b)
            return 0

        lax.fori_loop(0, nchunks // 2 - 1, body, 0)
        for b in range(2):
            wait_gathers(b)
            fire_scatters(b)
            drain_scatters(b)
        plsc.subcore_barrier()
        pltpu.sync_copy(acc.at[pl.ds(s * rpn, rpn)],
                        t_hbm.at[pl.ds(s * rpn, rpn)])

    @pl.when(c == 0)
    def _():
        do_quarter(h0, t0)
        do_quarter(h1, t1)

    @pl.when(c == 1)
    def _():
        do_quarter(h2, t2)
        do_quarter(h3, t3)


def _make_seg_sum(npad, ne_pad):
    nchunks = ne_pad // (NTILES * CH)
    rpn = npad // NTILES
    mesh = plsc.VectorSubcoreMesh(
        core_axis_name="c", subcore_axis_name="s",
        num_cores=2, num_subcores=NTILES)
    return functools.partial(
        pl.kernel,
        mesh=mesh,
        compiler_params=pltpu.CompilerParams(use_tc_tiling_on_sc=False),
        out_type=tuple(jax.ShapeDtypeStruct((npad, DQ), jnp.float32)
                       for _ in range(NQ)),
        scratch_types=[
            pltpu.VMEM((2, GPC, EG), jnp.int32),
            pltpu.VMEM((2, GPC, EG), jnp.int32),
            pltpu.VMEM((2, CH, DQ), jnp.float32),
            pltpu.VMEM_SHARED((npad, DQ), jnp.float32),
            pltpu.SemaphoreType.DMA,
            pltpu.SemaphoreType.DMA,
            pltpu.SemaphoreType.DMA,
            pltpu.SemaphoreType.DMA,
        ],
    )(functools.partial(_seg_sum_body, nchunks, rpn))


def _prep_tc(emb_blk, tdeg_blk, dinv_blk, h0_blk, h1_blk, h2_blk, h3_blk):
    deg = tdeg_blk[:, 0:1]
    dinv = jnp.where(deg > 0.0,
                     jax.lax.rsqrt(jnp.maximum(deg, 1e-12)), 0.0)
    dinvq = jnp.broadcast_to(dinv, (deg.shape[0], DQ))
    dinv_blk[...] = dinvq
    for q, h_blk in enumerate((h0_blk, h1_blk, h2_blk, h3_blk)):
        h_blk[...] = dinvq * emb_blk[:, q * DQ:(q + 1) * DQ]


def _finalize_tc(t0_blk, t1_blk, t2_blk, t3_blk, dinv_blk, tot_blk,
                 h0_blk, h1_blk, h2_blk, h3_blk, out_blk):
    dinvq = dinv_blk[...]
    hs = (h0_blk, h1_blk, h2_blk, h3_blk)
    for q, t_blk in enumerate((t0_blk, t1_blk, t2_blk, t3_blk)):
        g = dinvq * t_blk[...]
        hs[q][...] = dinvq * g
        out_blk[:, q * DQ:(q + 1) * DQ] = tot_blk[:, q * DQ:(q + 1) * DQ] + g


def _last_tc(t0_blk, t1_blk, t2_blk, t3_blk, dinv_blk, tot_blk, out_blk):
    dinvq = dinv_blk[...]
    for q, t_blk in enumerate((t0_blk, t1_blk, t2_blk, t3_blk)):
        out_blk[:, q * DQ:(q + 1) * DQ] = (
            tot_blk[:, q * DQ:(q + 1) * DQ] + dinvq * t_blk[...]) * 0.25


def kernel(user_emb, item_emb, user_idx, item_idx):
    n_users = user_emb.shape[0]
    n_items = item_emb.shape[0]
    n_nodes = n_users + n_items
    n_edges = user_idx.shape[0]
    dim = user_emb.shape[1]

    npad = ((n_nodes + 1 + 127) // 128) * 128
    ne = 2 * n_edges
    ne_pad = ((ne + NTILES * CH - 1) // (NTILES * CH)) * (NTILES * CH)
    rpn = npad // NTILES

    src = jnp.concatenate([user_idx, item_idx + n_users])
    dst = jnp.concatenate([item_idx + n_users, user_idx])
    pad = jnp.full((ne_pad - ne,), n_nodes, dtype=jnp.int32)
    src2 = jnp.concatenate([src, pad]).reshape(ne_pad // EG, EG)
    dst2 = jnp.concatenate([dst, pad]).reshape(ne_pad // EG, EG)

    emb = jnp.concatenate([user_emb, item_emb], axis=0)
    emb = jnp.concatenate(
        [emb, jnp.zeros((npad - n_nodes, dim), jnp.float32)], axis=0)
    onesq = jnp.ones((npad, DQ), jnp.float32)
    z = jnp.zeros((rpn, DQ), jnp.float32)

    seg_sum = _make_seg_sum(npad, ne_pad)

    tdeg = seg_sum(src2, dst2, onesq, onesq, onesq, onesq, z)[0]

    nblk = 16
    rb = npad // nblk
    row_specq = pl.BlockSpec((rb, DQ), lambda i: (i, 0))
    row_spec64 = pl.BlockSpec((rb, dim), lambda i: (i, 0))
    sdq = jax.ShapeDtypeStruct((npad, DQ), jnp.float32)
    dinvq, h0, h1, h2, h3 = pl.pallas_call(
        _prep_tc,
        grid=(nblk,),
        in_specs=[row_spec64, row_specq],
        out_specs=[row_specq] * 5,
        out_shape=[sdq] * 5,
    )(emb, tdeg)

    total = emb
    for layer in range(3):
        t0, t1, t2, t3 = seg_sum(src2, dst2, h0, h1, h2, h3, z)
        if layer < 2:
            h0, h1, h2, h3, total = pl.pallas_call(
                _finalize_tc,
                grid=(nblk,),
                in_specs=[row_specq] * 5 + [row_spec64],
                out_specs=[row_specq] * 4 + [row_spec64],
                out_shape=[sdq] * 4 +
                          [jax.ShapeDtypeStruct((npad, dim), jnp.float32)],
            )(t0, t1, t2, t3, dinvq, total)
        else:
            out = pl.pallas_call(
                _last_tc,
                grid=(nblk,),
                in_specs=[row_specq] * 5 + [row_spec64],
                out_specs=row_spec64,
                out_shape=jax.ShapeDtypeStruct((npad, dim), jnp.float32),
            )(t0, t1, t2, t3, dinvq, total)

    return out[:n_users], out[n_users:n_nodes]
